# baseline (speedup 1.0000x reference)
import functools

import jax
import jax.numpy as jnp
from jax import lax
from jax.experimental import pallas as pl
from jax.experimental.pallas import tpu as pltpu
from jax.experimental.pallas import tpu_sc as plsc

D_MODEL = 64
SCALE = float(D_MODEL) ** 0.5

B_ROWS = 4096
B_COLS = 200
B_TOTAL = B_ROWS * B_COLS

NUM_WORKERS = 32
CHUNK = 128
PER_WORKER = B_TOTAL // NUM_WORKERS
CHUNKS_PER_WORKER = PER_WORKER // CHUNK


def _sc_body(idx_hbm, table_hbm, out_hbm, idx_v, rows_v, sem):
    nc = 2
    wid = lax.axis_index("s") * nc + lax.axis_index("c")

    pltpu.sync_copy(idx_hbm.at[pl.ds(wid * CHUNKS_PER_WORKER, CHUNKS_PER_WORKER)], idx_v)

    base_row = wid * PER_WORKER

    @pl.loop(0, CHUNKS_PER_WORKER)
    def _chunk(c):
        pltpu.async_copy(table_hbm.at[idx_v.at[c]], rows_v, sem).wait()

        @pl.loop(0, CHUNK)
        def _row(i):
            for j in range(D_MODEL // 16):
                sl = pl.ds(j * 16, 16)
                rows_v[i, sl] = rows_v[i, sl] * SCALE

        pltpu.sync_copy(rows_v, out_hbm.at[pl.ds(base_row + c * CHUNK, CHUNK)])


@jax.jit
def _embed(x_flat2d, embedding):
    mesh = plsc.VectorSubcoreMesh(core_axis_name="c", subcore_axis_name="s")
    run = pl.kernel(
        _sc_body,
        out_type=jax.ShapeDtypeStruct((B_TOTAL, D_MODEL), jnp.float32),
        mesh=mesh,
        compiler_params=pltpu.CompilerParams(use_tc_tiling_on_sc=False),
        scratch_types=[
            pltpu.VMEM((CHUNKS_PER_WORKER, CHUNK), jnp.int32),
            pltpu.VMEM((CHUNK, D_MODEL), jnp.float32),
            pltpu.SemaphoreType.DMA,
        ],
    )
    return run(x_flat2d, embedding)


def kernel(x, embedding):
    x_flat2d = x.astype(jnp.int32).reshape(B_TOTAL // CHUNK, CHUNK)
    out = _embed(x_flat2d, embedding)
    return out.reshape(B_ROWS, B_COLS, D_MODEL)


# speedup vs baseline: 1.2130x; 1.2130x over previous
import jax
import jax.numpy as jnp
from jax import lax
from jax.experimental import pallas as pl
from jax.experimental.pallas import tpu as pltpu
from jax.experimental.pallas import tpu_sc as plsc

D_MODEL = 64
SCALE = float(D_MODEL) ** 0.5

B_ROWS = 4096
B_COLS = 200
B_TOTAL = B_ROWS * B_COLS

NUM_WORKERS = 32
IDX_CHUNK = 128
SUPER = 256
GPS = SUPER // IDX_CHUNK
PER_WORKER = B_TOTAL // NUM_WORKERS
IDX_ROWS = PER_WORKER // IDX_CHUNK
NSUP = PER_WORKER // SUPER


def _sc_body(idx_hbm, table_hbm, out_hbm, idx_v, g0, g1, o0, o1,
             gs0, gs1, os0, os1):
    nc = 2
    wid = lax.axis_index("s") * nc + lax.axis_index("c")

    gbuf = (g0, g1)
    obuf = (o0, o1)
    gsem = (gs0, gs1)
    osem = (os0, os1)

    pltpu.sync_copy(idx_hbm.at[pl.ds(wid * IDX_ROWS, IDX_ROWS)], idx_v)

    base_row = wid * PER_WORKER

    def start_gathers(g, b):
        for k in range(GPS):
            pltpu.async_copy(
                table_hbm.at[idx_v.at[g * GPS + k]],
                gbuf[b].at[pl.ds(k * IDX_CHUNK, IDX_CHUNK)],
                gsem[b])

    start_gathers(0, 0)
    start_gathers(1, 1)

    @pl.loop(0, NSUP // 2)
    def _outer(s):
        for b in range(2):
            g = s * 2 + b
            pltpu.make_async_copy(
                out_hbm.at[pl.ds(base_row, SUPER)], gbuf[b], gsem[b]).wait()

            @pl.when(g >= 2)
            def _():
                pltpu.make_async_copy(
                    obuf[b],
                    out_hbm.at[pl.ds(base_row + (g - 2) * SUPER, SUPER)],
                    osem[b]).wait()

            @plsc.parallel_loop(0, SUPER, unroll=4)
            def _row(i):
                for j in range(D_MODEL // 16):
                    sl = pl.ds(j * 16, 16)
                    obuf[b][i, sl] = gbuf[b][i, sl] * SCALE

            @pl.when(g + 2 < NSUP)
            def _():
                start_gathers(g + 2, b)

            pltpu.async_copy(
                obuf[b],
                out_hbm.at[pl.ds(base_row + g * SUPER, SUPER)],
                osem[b])

    for b in range(2):
        g = NSUP - 2 + b
        pltpu.make_async_copy(
            obuf[b],
            out_hbm.at[pl.ds(base_row + g * SUPER, SUPER)],
            osem[b]).wait()


@jax.jit
def _embed(x_flat2d, embedding):
    mesh = plsc.VectorSubcoreMesh(core_axis_name="c", subcore_axis_name="s")
    run = pl.kernel(
        _sc_body,
        out_type=jax.ShapeDtypeStruct((B_TOTAL, D_MODEL), jnp.float32),
        mesh=mesh,
        compiler_params=pltpu.CompilerParams(use_tc_tiling_on_sc=False),
        scratch_types=[
            pltpu.VMEM((IDX_ROWS, IDX_CHUNK), jnp.int32),
            pltpu.VMEM((SUPER, D_MODEL), jnp.float32),
            pltpu.VMEM((SUPER, D_MODEL), jnp.float32),
            pltpu.VMEM((SUPER, D_MODEL), jnp.float32),
            pltpu.VMEM((SUPER, D_MODEL), jnp.float32),
            pltpu.SemaphoreType.DMA,
            pltpu.SemaphoreType.DMA,
            pltpu.SemaphoreType.DMA,
            pltpu.SemaphoreType.DMA,
        ],
    )
    return run(x_flat2d, embedding)


def kernel(x, embedding):
    x_flat2d = x.astype(jnp.int32).reshape(B_TOTAL // IDX_CHUNK, IDX_CHUNK)
    out = _embed(x_flat2d, embedding)
    return out.reshape(B_ROWS, B_COLS, D_MODEL)
